# baseline (speedup 1.0000x reference)
import jax
import jax.numpy as jnp
from jax import lax
from jax.experimental import pallas as pl
from jax.experimental.pallas import tpu as pltpu
from jax.experimental.pallas import tpu_sc as plsc

_BATCH = 16384
_FIELDS = 26
_DIM = 32
_B = _BATCH * _FIELDS
_NE = 1000000

_NC = 2
_NS = 16
_NW = _NC * _NS

_GATHER = 128
_CHUNK = 1024
_G_PER_CHUNK = _CHUNK // _GATHER
_B_PER_W = _B // _NW
_N_CHUNKS = _B_PER_W // _CHUNK
_NBUF = 2


def _gather_body(idx_hbm, table_hbm, out_hbm, idx_v, rows_v, isem, gsem, osem):
    wid = lax.axis_index("s") * _NC + lax.axis_index("c")
    grp0 = wid * (_B_PER_W // _GATHER)

    def idx_cp(c):
        slot = c % _NBUF
        return pltpu.make_async_copy(
            idx_hbm.at[pl.ds(grp0 + c * _G_PER_CHUNK, _G_PER_CHUNK)],
            idx_v.at[slot], isem.at[slot])

    def out_cp(c):
        slot = c % _NBUF
        return pltpu.make_async_copy(
            rows_v.at[slot],
            out_hbm.at[pl.ds((grp0 + c * _G_PER_CHUNK) * _GATHER, _CHUNK)],
            osem.at[slot])

    def gather_cp(c, j):
        slot = c % _NBUF
        return pltpu.make_async_copy(
            table_hbm.at[idx_v.at[slot].at[j]],
            rows_v.at[slot].at[pl.ds(j * _GATHER, _GATHER)],
            gsem.at[slot])

    for p in range(_NBUF):
        idx_cp(p).start()

    for c in range(_N_CHUNKS + 1):
        if c < _N_CHUNKS:
            idx_cp(c).wait()
            if c >= _NBUF:
                out_cp(c - _NBUF).wait()
            for j in range(_G_PER_CHUNK):
                gather_cp(c, j).start()
        if c >= 1:
            for j in range(_G_PER_CHUNK):
                gather_cp(c - 1, j).wait()
            out_cp(c - 1).start()
            if c - 1 + _NBUF < _N_CHUNKS:
                idx_cp(c - 1 + _NBUF).start()

    for c in range(_N_CHUNKS - _NBUF, _N_CHUNKS):
        out_cp(c).wait()


_NBLK = _NE // 128
_REM = _NE - _NBLK * 128


def _relayout_body(wt_hbm, tail_hbm, out_hbm, in_v, out_v, isem, osem):
    wid = lax.axis_index("s") * _NC + lax.axis_index("c")
    iota = lax.iota(jnp.int32, 16)
    r0 = iota
    r1 = iota + 16
    nblk = (_NBLK - 1 - wid) // _NW + 1

    def in_cp(t):
        c = wid + _NW * t
        return pltpu.make_async_copy(
            wt_hbm.at[:, pl.ds(c * 128, 128)],
            in_v.at[t % 2].at[:, pl.ds(0, 128)], isem)

    def out_cp(t):
        c = wid + _NW * t
        return pltpu.make_async_copy(
            out_v.at[t % 2], out_hbm.at[pl.ds(c * 32, 32)], osem)

    def transpose_block(src, dst, nrows, col0=0):
        for g0 in range(0, nrows, 2):
            vals = []
            for g in (g0, g0 + 1):
                for u in range(8):
                    row_idx = r0 if u % 2 == 0 else r1
                    col_idx = jnp.full((16,), col0 + 4 * g + u // 2,
                                       jnp.int32)
                    vals.append(plsc.load_gather(src, [row_idx, col_idx]))
            for k, v in enumerate(vals):
                dst[g0 + k // 8, pl.ds(16 * (k % 8), 16)] = v

    in_cp(0).start()

    def step(t, carry):
        @pl.when(t + 1 < nblk)
        def _():
            in_cp(t + 1).start()
        in_cp(t).wait()

        @pl.when(t >= 2)
        def _():
            out_cp(t - 2).wait()
        transpose_block(in_v.at[t % 2], out_v.at[t % 2], 32)
        out_cp(t).start()
        return carry

    lax.fori_loop(0, nblk, step, 0)
    out_cp(nblk - 2).wait()
    out_cp(nblk - 1).wait()

    @pl.when(wid == 4)
    def _():
        nrem = _REM * _DIM // 128
        pltpu.sync_copy(tail_hbm, out_v.at[0].at[pl.ds(0, nrem)])
        pltpu.sync_copy(out_v.at[0].at[pl.ds(0, nrem)],
                        out_hbm.at[pl.ds(_NBLK * 32, nrem)])


def _sc_mesh():
    return plsc.VectorSubcoreMesh(
        core_axis_name="c", subcore_axis_name="s",
        num_cores=_NC, num_subcores=_NS,
    )


@jax.jit
def kernel(x, weight):
    idx2d = x.reshape(_B // _GATHER, _GATHER).astype(jnp.int32)
    wt = weight.T
    tail = weight[_NBLK * 128:].reshape(_REM * _DIM // 128, 128)

    w128 = pl.kernel(
        _relayout_body,
        out_type=jax.ShapeDtypeStruct((_NE * _DIM // 128, 128), jnp.float32),
        mesh=_sc_mesh(),
        scratch_types=[
            pltpu.VMEM((2, _DIM, 129), jnp.float32),
            pltpu.VMEM((2, _DIM, 128), jnp.float32),
            pltpu.SemaphoreType.DMA,
            pltpu.SemaphoreType.DMA,
        ],
        compiler_params=pltpu.CompilerParams(
            use_tc_tiling_on_sc=True, needs_layout_passes=False),
    )(wt, tail)
    w32 = w128.reshape(_NE, _DIM)

    out_flat = pl.kernel(
        _gather_body,
        out_type=jax.ShapeDtypeStruct((_B, _DIM), jnp.float32),
        mesh=_sc_mesh(),
        scratch_types=[
            pltpu.VMEM((_NBUF, _G_PER_CHUNK, _GATHER), jnp.int32),
            pltpu.VMEM((_NBUF, _CHUNK, _DIM), jnp.float32),
            pltpu.SemaphoreType.DMA((_NBUF,)),
            pltpu.SemaphoreType.DMA((_NBUF,)),
            pltpu.SemaphoreType.DMA((_NBUF,)),
        ],
        compiler_params=pltpu.CompilerParams(use_tc_tiling_on_sc=False),
    )(idx2d, w32)
    return out_flat.reshape(_BATCH, _FIELDS, _DIM)


# speedup vs baseline: 1.0485x; 1.0485x over previous
import jax
import jax.numpy as jnp
from jax import lax
from jax.experimental import pallas as pl
from jax.experimental.pallas import tpu as pltpu
from jax.experimental.pallas import tpu_sc as plsc

_BATCH = 16384
_FIELDS = 26
_DIM = 32
_B = _BATCH * _FIELDS
_NE = 1000000

_NC = 2
_NS = 16
_NW = _NC * _NS

_GATHER = 128
_CHUNK = 1024
_G_PER_CHUNK = _CHUNK // _GATHER
_B_PER_W = _B // _NW
_N_CHUNKS = _B_PER_W // _CHUNK
_NBUF = 2


def _gather_body(idx_hbm, table_hbm, out_hbm, idx_v, rows_v, isem, gsem, osem):
    wid = lax.axis_index("s") * _NC + lax.axis_index("c")
    grp0 = wid * (_B_PER_W // _GATHER)

    def idx_cp(c):
        slot = c % _NBUF
        return pltpu.make_async_copy(
            idx_hbm.at[pl.ds(grp0 + c * _G_PER_CHUNK, _G_PER_CHUNK)],
            idx_v.at[slot], isem.at[slot])

    def out_cp(c):
        slot = c % _NBUF
        return pltpu.make_async_copy(
            rows_v.at[slot],
            out_hbm.at[pl.ds((grp0 + c * _G_PER_CHUNK) * _GATHER, _CHUNK)],
            osem.at[slot])

    def gather_cp(c, j):
        slot = c % _NBUF
        return pltpu.make_async_copy(
            table_hbm.at[idx_v.at[slot].at[j]],
            rows_v.at[slot].at[pl.ds(j * _GATHER, _GATHER)],
            gsem.at[slot])

    for p in range(_NBUF):
        idx_cp(p).start()

    for c in range(_N_CHUNKS + 1):
        if c < _N_CHUNKS:
            idx_cp(c).wait()
            if c >= _NBUF:
                out_cp(c - _NBUF).wait()
            for j in range(_G_PER_CHUNK):
                gather_cp(c, j).start()
        if c >= 1:
            for j in range(_G_PER_CHUNK):
                gather_cp(c - 1, j).wait()
            out_cp(c - 1).start()
            if c - 1 + _NBUF < _N_CHUNKS:
                idx_cp(c - 1 + _NBUF).start()

    for c in range(_N_CHUNKS - _NBUF, _N_CHUNKS):
        out_cp(c).wait()


_NBLK = _NE // 128
_REM = _NE - _NBLK * 128


def _relayout_body(wt_hbm, tail_hbm, out_hbm, in_v, out_v, isem, osem):
    wid = lax.axis_index("s") * _NC + lax.axis_index("c")
    iota = lax.iota(jnp.int32, 16)
    r0 = iota
    r1 = iota + 16
    nblk = (_NBLK - 1 - wid) // _NW + 1

    def in_cp(t):
        c = wid + _NW * t
        return pltpu.make_async_copy(
            wt_hbm.at[:, pl.ds(c * 128, 128)], in_v.at[t % 4], isem)

    def out_cp(t):
        c = wid + _NW * t
        return pltpu.make_async_copy(
            out_v.at[t % 4], out_hbm.at[pl.ds(c * 32, 32)], osem)

    def transpose_block(src, dst, nrows, col0=0):
        for g0 in range(0, nrows, 2):
            vals = []
            for g in (g0, g0 + 1):
                for u in range(8):
                    row_idx = r0 if u % 2 == 0 else r1
                    col_idx = jnp.full((16,), col0 + 4 * g + u // 2,
                                       jnp.int32)
                    vals.append(plsc.load_gather(src, [row_idx, col_idx]))
            for k, v in enumerate(vals):
                dst[g0 + k // 8, pl.ds(16 * (k % 8), 16)] = v

    for p in range(3):
        in_cp(p).start()

    def step(t, carry):
        @pl.when(t + 3 < nblk)
        def _():
            in_cp(t + 3).start()
        in_cp(t).wait()

        @pl.when(t >= 4)
        def _():
            out_cp(t - 4).wait()
        transpose_block(in_v.at[t % 4], out_v.at[t % 4], 32)
        out_cp(t).start()
        return carry

    lax.fori_loop(0, nblk, step, 0)
    for p in range(4):
        out_cp(nblk - 4 + p).wait()

    @pl.when(wid == 4)
    def _():
        nrem = _REM * _DIM // 128
        pltpu.sync_copy(tail_hbm, out_v.at[0].at[pl.ds(0, nrem)])
        pltpu.sync_copy(out_v.at[0].at[pl.ds(0, nrem)],
                        out_hbm.at[pl.ds(_NBLK * 32, nrem)])


def _sc_mesh():
    return plsc.VectorSubcoreMesh(
        core_axis_name="c", subcore_axis_name="s",
        num_cores=_NC, num_subcores=_NS,
    )


@jax.jit
def kernel(x, weight):
    idx2d = x.reshape(_B // _GATHER, _GATHER).astype(jnp.int32)
    wt = weight.T
    tail = weight[_NBLK * 128:].reshape(_REM * _DIM // 128, 128)

    w128 = pl.kernel(
        _relayout_body,
        out_type=jax.ShapeDtypeStruct((_NE * _DIM // 128, 128), jnp.float32),
        mesh=_sc_mesh(),
        scratch_types=[
            pltpu.VMEM((4, _DIM, 128), jnp.float32),
            pltpu.VMEM((4, _DIM, 128), jnp.float32),
            pltpu.SemaphoreType.DMA,
            pltpu.SemaphoreType.DMA,
        ],
        compiler_params=pltpu.CompilerParams(
            use_tc_tiling_on_sc=True, needs_layout_passes=False),
    )(wt, tail)
    w32 = w128.reshape(_NE, _DIM)

    out_flat = pl.kernel(
        _gather_body,
        out_type=jax.ShapeDtypeStruct((_B, _DIM), jnp.float32),
        mesh=_sc_mesh(),
        scratch_types=[
            pltpu.VMEM((_NBUF, _G_PER_CHUNK, _GATHER), jnp.int32),
            pltpu.VMEM((_NBUF, _CHUNK, _DIM), jnp.float32),
            pltpu.SemaphoreType.DMA((_NBUF,)),
            pltpu.SemaphoreType.DMA((_NBUF,)),
            pltpu.SemaphoreType.DMA((_NBUF,)),
        ],
        compiler_params=pltpu.CompilerParams(use_tc_tiling_on_sc=False),
    )(idx2d, w32)
    return out_flat.reshape(_BATCH, _FIELDS, _DIM)


# speedup vs baseline: 1.0761x; 1.0264x over previous
import jax
import jax.numpy as jnp
from jax import lax
from jax.experimental import pallas as pl
from jax.experimental.pallas import tpu as pltpu
from jax.experimental.pallas import tpu_sc as plsc

_BATCH = 16384
_FIELDS = 26
_DIM = 32
_B = _BATCH * _FIELDS

_NC = 2
_NS = 16
_NW = _NC * _NS

_GATHER = 128
_CHUNK = 1024
_G_PER_CHUNK = _CHUNK // _GATHER
_B_PER_W = _B // _NW
_N_CHUNKS = _B_PER_W // _CHUNK
_NBUF = 2


def _gather_body(idx_hbm, table_hbm, out_hbm, idx_v, rows_v, isem, gsem, osem):
    wid = lax.axis_index("s") * _NC + lax.axis_index("c")
    grp0 = wid * (_B_PER_W // _GATHER)

    def idx_cp(c):
        slot = c % _NBUF
        return pltpu.make_async_copy(
            idx_hbm.at[pl.ds(grp0 + c * _G_PER_CHUNK, _G_PER_CHUNK)],
            idx_v.at[slot], isem.at[slot])

    def out_cp(c):
        slot = c % _NBUF
        return pltpu.make_async_copy(
            rows_v.at[slot],
            out_hbm.at[pl.ds((grp0 + c * _G_PER_CHUNK) * _GATHER, _CHUNK)],
            osem.at[slot])

    def gather_cp(c, j):
        slot = c % _NBUF
        return pltpu.make_async_copy(
            table_hbm.at[idx_v.at[slot].at[j]],
            rows_v.at[slot].at[pl.ds(j * _GATHER, _GATHER)],
            gsem.at[slot])

    for p in range(_NBUF):
        idx_cp(p).start()

    for c in range(_N_CHUNKS + 1):
        if c < _N_CHUNKS:
            idx_cp(c).wait()
            if c >= _NBUF:
                out_cp(c - _NBUF).wait()
            for j in range(_G_PER_CHUNK):
                gather_cp(c, j).start()
        if c >= 1:
            for j in range(_G_PER_CHUNK):
                gather_cp(c - 1, j).wait()
            out_cp(c - 1).start()
            if c - 1 + _NBUF < _N_CHUNKS:
                idx_cp(c - 1 + _NBUF).start()

    for c in range(_N_CHUNKS - _NBUF, _N_CHUNKS):
        out_cp(c).wait()


@jax.jit
def kernel(x, weight):
    idx2d = x.reshape(_B // _GATHER, _GATHER).astype(jnp.int32)
    w128 = jax.lax.optimization_barrier(weight.reshape(250000, 128))
    w32 = w128.reshape(1000000, _DIM)
    mesh = plsc.VectorSubcoreMesh(
        core_axis_name="c", subcore_axis_name="s",
        num_cores=_NC, num_subcores=_NS,
    )
    out_flat = pl.kernel(
        _gather_body,
        out_type=jax.ShapeDtypeStruct((_B, _DIM), jnp.float32),
        mesh=mesh,
        scratch_types=[
            pltpu.VMEM((_NBUF, _G_PER_CHUNK, _GATHER), jnp.int32),
            pltpu.VMEM((_NBUF, _CHUNK, _DIM), jnp.float32),
            pltpu.SemaphoreType.DMA((_NBUF,)),
            pltpu.SemaphoreType.DMA((_NBUF,)),
            pltpu.SemaphoreType.DMA((_NBUF,)),
        ],
        compiler_params=pltpu.CompilerParams(use_tc_tiling_on_sc=False),
    )(idx2d, w32)
    out128 = jax.lax.optimization_barrier(out_flat.reshape(_B * _DIM // 128, 128))
    return out128.reshape(_BATCH, _FIELDS, _DIM)
